# baseline (speedup 1.0000x reference)
import jax
import jax.numpy as jnp
from jax import lax
from jax.experimental import pallas as pl
from jax.experimental.pallas import tpu as pltpu
from jax.experimental.pallas import tpu_sc as plsc

_NE = 8
_D = 2048
_DFF = 1024
_EPS = 1e-6
_SCALE = _NE / 2

_T = 2048
_RT = 256
_R = 512
_NBLK = 2 * _T // _R + _NE
_FC = 512
_NC = _DFF // _FC
_PAD = _NBLK * _R

_SC_NC = 2
_NW = _SC_NC * 16
_CHUNK = _T // _NW
_NCH = _CHUNK // 16
_WSW = 128
_CW = 8


def _router_body(x_ref, nw_ref, rw_ref, i1_ref, i2_ref, r1_ref, r2_ref,
                 w1_ref, w2_ref, cnt_ref, cnt_sc):
    tile = pl.program_id(0)

    @pl.when(tile == 0)
    def _():
        cnt_sc[...] = jnp.zeros_like(cnt_sc)

    x = x_ref[...]
    var = jnp.mean(x * x, axis=-1, keepdims=True)
    xn = (x * jax.lax.rsqrt(var + _EPS)) * nw_ref[...]
    logits = jax.lax.dot_general(
        xn, rw_ref[...], (((1,), (1,)), ((), ())),
        preferred_element_type=jnp.float32)
    m = jnp.max(logits, axis=-1, keepdims=True)
    p = jnp.exp(logits - m)
    probs = p / jnp.sum(p, axis=-1, keepdims=True)

    eio = jax.lax.broadcasted_iota(jnp.int32, probs.shape, 1)
    w1 = jnp.max(probs, axis=-1, keepdims=True)
    i1 = jnp.min(jnp.where(probs == w1, eio, _NE), axis=-1, keepdims=True)
    masked = jnp.where(eio == i1, -1.0, probs)
    w2 = jnp.max(masked, axis=-1, keepdims=True)
    i2 = jnp.min(jnp.where(masked == w2, eio, _NE), axis=-1, keepdims=True)
    wsum = w1 + w2
    w1_ref[...] = jnp.broadcast_to(w1 * (_SCALE / wsum), (_RT, _WSW))
    w2_ref[...] = jnp.broadcast_to(w2 * (_SCALE / wsum), (_RT, _WSW))
    i1_ref[...] = i1
    i2_ref[...] = i2

    m1 = (eio == i1).astype(jnp.float32)
    m2 = (eio == i2).astype(jnp.float32)
    msum = m1 + m2
    rio = jax.lax.broadcasted_iota(jnp.int32, (_RT, _RT), 0)
    cio = jax.lax.broadcasted_iota(jnp.int32, (_RT, _RT), 1)
    tril = (cio < rio).astype(jnp.float32)
    excl = jax.lax.dot_general(tril, msum, (((1,), (0,)), ((), ())),
                               preferred_element_type=jnp.float32)
    base = excl + cnt_sc[...]
    r1_ref[...] = jnp.sum(base * m1, axis=-1, keepdims=True).astype(jnp.int32)
    r2_ref[...] = jnp.sum(base * m2, axis=-1, keepdims=True).astype(jnp.int32)
    cnt_sc[...] += jnp.sum(msum, axis=0, keepdims=True)
    cnt_ref[...] = cnt_sc[...].astype(jnp.int32)


def _router(flat, norm_w, router_w):
    col_i = jax.ShapeDtypeStruct((_T, 1), jnp.int32)
    col_f = jax.ShapeDtypeStruct((_T, _WSW), jnp.float32)
    col_spec = pl.BlockSpec((_RT, 1), lambda i: (i, 0))
    wide_spec = pl.BlockSpec((_RT, _WSW), lambda i: (i, 0))
    return pl.pallas_call(
        _router_body,
        grid=(_T // _RT,),
        in_specs=[
            pl.BlockSpec((_RT, _D), lambda i: (i, 0)),
            pl.BlockSpec((1, _D), lambda i: (0, 0)),
            pl.BlockSpec((_NE, _D), lambda i: (0, 0)),
        ],
        out_specs=[col_spec, col_spec, col_spec, col_spec, wide_spec,
                   wide_spec, pl.BlockSpec((1, _NE), lambda i: (0, 0))],
        out_shape=[col_i, col_i, col_i, col_i, col_f, col_f,
                   jax.ShapeDtypeStruct((1, _NE), jnp.int32)],
        scratch_shapes=[pltpu.VMEM((1, _NE), jnp.float32)],
    )(flat, norm_w.reshape(1, _D), router_w)


def _dispatch_body(flat_h, i1_h, i2_h, r1_h, r2_h, w1_h, w2_h, offs_h,
                   xs_h, ws_h, p1_h, p2_h,
                   offs_v, i1_v, i2_v, r1_v, r2_v, w1_v, w2_v, p1_v, p2_v,
                   rows_v, sem_g, sem_s, sem_w):
    wid = lax.axis_index("s") * _SC_NC + lax.axis_index("c")
    base = wid * _CHUNK
    pltpu.sync_copy(offs_h, offs_v)
    pltpu.sync_copy(i1_h.at[pl.ds(base, _CHUNK)], i1_v)
    pltpu.sync_copy(i2_h.at[pl.ds(base, _CHUNK)], i2_v)
    pltpu.sync_copy(r1_h.at[pl.ds(base, _CHUNK)], r1_v)
    pltpu.sync_copy(r2_h.at[pl.ds(base, _CHUNK)], r2_v)
    pltpu.sync_copy(w1_h.at[pl.ds(base, _CHUNK)], w1_v)
    pltpu.sync_copy(w2_h.at[pl.ds(base, _CHUNK)], w2_v)
    for c in range(_NCH):
        sl = pl.ds(c * 16, 16)
        p1_v[sl] = plsc.load_gather(offs_v, [i1_v[sl]]) + r1_v[sl]
        p2_v[sl] = plsc.load_gather(offs_v, [i2_v[sl]]) + r2_v[sl]
    pltpu.sync_copy(p1_v, p1_h.at[pl.ds(base, _CHUNK)])
    pltpu.sync_copy(p2_v, p2_h.at[pl.ds(base, _CHUNK)])
    wd = []
    for c in range(_NCH):
        sl = pl.ds(c * 16, 16)
        wd.append(pltpu.async_copy(w1_v.at[sl], ws_h.at[p1_v[sl]], sem_w))
        wd.append(pltpu.async_copy(w2_v.at[sl], ws_h.at[p2_v[sl]], sem_w))
    def tok(c):
        return (base + c * 16
                + jax.lax.broadcasted_iota(jnp.int32, (16,), 0))

    g = pltpu.async_copy(flat_h.at[tok(0)], rows_v.at[0], sem_g)
    s_prev = []
    for c in range(_NCH):
        cur = c % 2
        sl = pl.ds(c * 16, 16)
        g.wait()
        for s in s_prev:
            s.wait()
        if c + 1 < _NCH:
            g = pltpu.async_copy(flat_h.at[tok(c + 1)], rows_v.at[1 - cur],
                                 sem_g)
        s_prev = [
            pltpu.async_copy(rows_v.at[cur], xs_h.at[p1_v[sl]], sem_s),
            pltpu.async_copy(rows_v.at[cur], xs_h.at[p2_v[sl]], sem_s),
        ]
    for s in s_prev:
        s.wait()
    for w in wd:
        w.wait()


def _dispatch(flat, i1, i2, r1, r2, w1, w2, offs16):
    return pl.kernel(
        _dispatch_body,
        out_type=[
            jax.ShapeDtypeStruct((_PAD, _D), jnp.float32),
            jax.ShapeDtypeStruct((_PAD, _WSW), jnp.float32),
            jax.ShapeDtypeStruct((_T,), jnp.int32),
            jax.ShapeDtypeStruct((_T,), jnp.int32),
        ],
        mesh=plsc.VectorSubcoreMesh(core_axis_name="c", subcore_axis_name="s"),
        compiler_params=pltpu.CompilerParams(needs_layout_passes=False),
        scratch_types=[
            pltpu.VMEM((16,), jnp.int32),
            pltpu.VMEM((_CHUNK,), jnp.int32),
            pltpu.VMEM((_CHUNK,), jnp.int32),
            pltpu.VMEM((_CHUNK,), jnp.int32),
            pltpu.VMEM((_CHUNK,), jnp.int32),
            pltpu.VMEM((_CHUNK, _WSW), jnp.float32),
            pltpu.VMEM((_CHUNK, _WSW), jnp.float32),
            pltpu.VMEM((_CHUNK,), jnp.int32),
            pltpu.VMEM((_CHUNK,), jnp.int32),
            pltpu.VMEM((2, 16, _D), jnp.float32),
            pltpu.SemaphoreType.DMA,
            pltpu.SemaphoreType.DMA,
            pltpu.SemaphoreType.DMA,
        ],
    )(flat, i1, i2, r1, r2, w1, w2, offs16)


def _gmm_body(be_ref, ntot_ref, xs_ref, g_ref, u_ref, d_ref, ws_ref, o_ref):
    b = pl.program_id(0)

    @pl.when(b < ntot_ref[0])
    def _():
        x = xs_ref[...].astype(jnp.bfloat16)
        g = jax.lax.dot_general(x, g_ref[0], (((1,), (1,)), ((), ())),
                                preferred_element_type=jnp.float32)
        u = jax.lax.dot_general(x, u_ref[0], (((1,), (1,)), ((), ())),
                                preferred_element_type=jnp.float32)
        h = ((g * jax.nn.sigmoid(g)) * u).astype(jnp.bfloat16)
        part = jax.lax.dot_general(h, d_ref[0], (((1,), (1,)), ((), ())),
                                   preferred_element_type=jnp.float32)
        o_ref[...] = part * ws_ref[:, :1]


def _bb(b, ntot):
    return jnp.minimum(b, ntot[0] - 1)


def _gmm(be16, ntot1, xs, gate_w, up_w, down_w, ws):
    grid_spec = pltpu.PrefetchScalarGridSpec(
        num_scalar_prefetch=2,
        grid=(_NBLK,),
        in_specs=[
            pl.BlockSpec((_R, _D), lambda b, be, nt: (_bb(b, nt), 0)),
            pl.BlockSpec((1, _DFF, _D),
                         lambda b, be, nt: (be[_bb(b, nt)], 0, 0)),
            pl.BlockSpec((1, _DFF, _D),
                         lambda b, be, nt: (be[_bb(b, nt)], 0, 0)),
            pl.BlockSpec((1, _D, _DFF),
                         lambda b, be, nt: (be[_bb(b, nt)], 0, 0)),
            pl.BlockSpec((_R, _WSW), lambda b, be, nt: (_bb(b, nt), 0)),
        ],
        out_specs=pl.BlockSpec((_R, _D), lambda b, be, nt: (_bb(b, nt), 0)),
    )
    return pl.pallas_call(
        _gmm_body,
        grid_spec=grid_spec,
        out_shape=jax.ShapeDtypeStruct((_PAD, _D), jnp.float32),
        compiler_params=pltpu.CompilerParams(
            dimension_semantics=("arbitrary",)),
    )(be16, ntot1, xs, gate_w, up_w, down_w, ws)


def _combine_body(ys_h, p1_h, p2_h, out_h,
                  p1_v, p2_v, a_v, b_v, o_v, sem_a, sem_b, sem_o):
    wid = lax.axis_index("s") * _SC_NC + lax.axis_index("c")
    base = wid * _CHUNK
    pltpu.sync_copy(p1_h.at[pl.ds(base, _CHUNK)], p1_v)
    pltpu.sync_copy(p2_h.at[pl.ds(base, _CHUNK)], p2_v)
    nch = _CHUNK // _CW
    ga = pltpu.async_copy(ys_h.at[p1_v.at[pl.ds(0, _CW)]], a_v.at[0], sem_a)
    gb = pltpu.async_copy(ys_h.at[p2_v.at[pl.ds(0, _CW)]], b_v.at[0], sem_b)
    od = [None, None]
    for c in range(nch):
        cur = c % 2
        ga.wait()
        gb.wait()
        if c + 1 < nch:
            sl = pl.ds((c + 1) * _CW, _CW)
            ga = pltpu.async_copy(ys_h.at[p1_v.at[sl]], a_v.at[1 - cur],
                                  sem_a)
            gb = pltpu.async_copy(ys_h.at[p2_v.at[sl]], b_v.at[1 - cur],
                                  sem_b)
        if od[cur] is not None:
            od[cur].wait()
        for r in range(_CW):
            @plsc.parallel_loop(0, _D // 16, unroll=8)
            def _(j, cur=cur, r=r):
                cs = pl.ds(j * 16, 16)
                o_v[cur, r, cs] = a_v[cur, r, cs] + b_v[cur, r, cs]

        od[cur] = pltpu.async_copy(o_v.at[cur],
                                   out_h.at[pl.ds(base + c * _CW, _CW)],
                                   sem_o)
    for d in od:
        if d is not None:
            d.wait()


def _combine(ys, p1, p2):
    return pl.kernel(
        _combine_body,
        out_type=jax.ShapeDtypeStruct((_T, _D), jnp.float32),
        mesh=plsc.VectorSubcoreMesh(core_axis_name="c", subcore_axis_name="s"),
        compiler_params=pltpu.CompilerParams(needs_layout_passes=False),
        scratch_types=[
            pltpu.VMEM((_CHUNK,), jnp.int32),
            pltpu.VMEM((_CHUNK,), jnp.int32),
            pltpu.VMEM((2, _CW, _D), jnp.float32),
            pltpu.VMEM((2, _CW, _D), jnp.float32),
            pltpu.VMEM((2, _CW, _D), jnp.float32),
            pltpu.SemaphoreType.DMA,
            pltpu.SemaphoreType.DMA,
            pltpu.SemaphoreType.DMA,
        ],
    )(ys, p1, p2)


def kernel(hidden_states, router_norm_w, router_w, gate_w, up_w, down_w):
    b, s, d = hidden_states.shape
    flat = hidden_states.reshape(b * s, d)
    i1, i2, r1, r2, w1, w2, counts = _router(flat, router_norm_w, router_w)
    cnt = counts[0]
    ntil = (cnt + _R - 1) // _R
    bounds = jnp.cumsum(ntil).astype(jnp.int32)
    boff = jnp.concatenate([jnp.zeros((1,), jnp.int32), bounds[:-1]])
    ntot1 = bounds[-1:]
    be16 = jnp.minimum(
        jnp.sum((bounds[None, :] <= jnp.arange(_NBLK)[:, None]).astype(
            jnp.int32), axis=1), _NE - 1).astype(jnp.int32)
    offs16 = jnp.zeros((16,), jnp.int32).at[:_NE].set(boff * _R)
    xs, ws, p1, p2 = _dispatch(flat, i1.reshape(_T), i2.reshape(_T),
                               r1.reshape(_T), r2.reshape(_T), w1, w2, offs16)
    ys = _gmm(be16, ntot1, xs, gate_w.astype(jnp.bfloat16),
              up_w.astype(jnp.bfloat16), down_w.astype(jnp.bfloat16), ws)
    out = _combine(ys, p1, p2)
    aux = jnp.array(0.0, dtype=jnp.float32)
    return (out.reshape(b, s, d), aux)


# speedup vs baseline: 1.4721x; 1.4721x over previous
import jax
import jax.numpy as jnp
from jax import lax
from jax.experimental import pallas as pl
from jax.experimental.pallas import tpu as pltpu
from jax.experimental.pallas import tpu_sc as plsc

_NE = 8
_D = 2048
_DFF = 1024
_EPS = 1e-6
_SCALE = _NE / 2

_T = 2048
_RT = 256
_R = 576
_NBLK = 16
_FC = 512
_NC = _DFF // _FC
_PAD = _NBLK * _R

_SC_NC = 2
_NW = _SC_NC * 16
_CHUNK = _T // _NW
_NCH = _CHUNK // 16
_WSW = 128
_CW = 8


def _router_body(x_ref, nw_ref, rw_ref, i1_ref, i2_ref, r1_ref, r2_ref,
                 w1_ref, w2_ref, cnt_ref, cnt_sc):
    tile = pl.program_id(0)

    @pl.when(tile == 0)
    def _():
        cnt_sc[...] = jnp.zeros_like(cnt_sc)

    x = x_ref[...]
    var = jnp.mean(x * x, axis=-1, keepdims=True)
    xn = (x * jax.lax.rsqrt(var + _EPS)) * nw_ref[...]
    logits = jax.lax.dot_general(
        xn, rw_ref[...], (((1,), (1,)), ((), ())),
        preferred_element_type=jnp.float32)
    m = jnp.max(logits, axis=-1, keepdims=True)
    p = jnp.exp(logits - m)
    probs = p / jnp.sum(p, axis=-1, keepdims=True)

    eio = jax.lax.broadcasted_iota(jnp.int32, probs.shape, 1)
    w1 = jnp.max(probs, axis=-1, keepdims=True)
    i1 = jnp.min(jnp.where(probs == w1, eio, _NE), axis=-1, keepdims=True)
    masked = jnp.where(eio == i1, -1.0, probs)
    w2 = jnp.max(masked, axis=-1, keepdims=True)
    i2 = jnp.min(jnp.where(masked == w2, eio, _NE), axis=-1, keepdims=True)
    wsum = w1 + w2
    w1_ref[...] = jnp.broadcast_to(w1 * (_SCALE / wsum), (_RT, _WSW))
    w2_ref[...] = jnp.broadcast_to(w2 * (_SCALE / wsum), (_RT, _WSW))
    i1_ref[...] = i1
    i2_ref[...] = i2

    m1 = (eio == i1).astype(jnp.float32)
    m2 = (eio == i2).astype(jnp.float32)
    msum = m1 + m2
    rio = jax.lax.broadcasted_iota(jnp.int32, (_RT, _RT), 0)
    cio = jax.lax.broadcasted_iota(jnp.int32, (_RT, _RT), 1)
    tril = (cio < rio).astype(jnp.float32)
    excl = jax.lax.dot_general(tril, msum, (((1,), (0,)), ((), ())),
                               preferred_element_type=jnp.float32)
    base = excl + cnt_sc[...]
    r1_ref[...] = jnp.sum(base * m1, axis=-1, keepdims=True).astype(jnp.int32)
    r2_ref[...] = jnp.sum(base * m2, axis=-1, keepdims=True).astype(jnp.int32)
    cnt_sc[...] += jnp.sum(msum, axis=0, keepdims=True)
    cnt_ref[...] = cnt_sc[...].astype(jnp.int32)


def _router(flat, norm_w, router_w):
    col_i = jax.ShapeDtypeStruct((_T, 1), jnp.int32)
    col_f = jax.ShapeDtypeStruct((_T, _WSW), jnp.float32)
    col_spec = pl.BlockSpec((_RT, 1), lambda i: (i, 0))
    wide_spec = pl.BlockSpec((_RT, _WSW), lambda i: (i, 0))
    return pl.pallas_call(
        _router_body,
        grid=(_T // _RT,),
        in_specs=[
            pl.BlockSpec((_RT, _D), lambda i: (i, 0)),
            pl.BlockSpec((1, _D), lambda i: (0, 0)),
            pl.BlockSpec((_NE, _D), lambda i: (0, 0)),
        ],
        out_specs=[col_spec, col_spec, col_spec, col_spec, wide_spec,
                   wide_spec, pl.BlockSpec((1, _NE), lambda i: (0, 0))],
        out_shape=[col_i, col_i, col_i, col_i, col_f, col_f,
                   jax.ShapeDtypeStruct((1, _NE), jnp.int32)],
        scratch_shapes=[pltpu.VMEM((1, _NE), jnp.float32)],
    )(flat, norm_w.reshape(1, _D), router_w)


def _dispatch_body(flat_h, i1_h, i2_h, r1_h, r2_h, w1_h, w2_h, offs_h,
                   xs_h, ws_h, p1_h, p2_h,
                   offs_v, i1_v, i2_v, r1_v, r2_v, w1_v, w2_v, p1_v, p2_v,
                   rows_v, sem_g, sem_s, sem_w):
    wid = lax.axis_index("s") * _SC_NC + lax.axis_index("c")
    base = wid * _CHUNK
    pltpu.sync_copy(offs_h, offs_v)
    pltpu.sync_copy(i1_h.at[pl.ds(base, _CHUNK)], i1_v)
    pltpu.sync_copy(i2_h.at[pl.ds(base, _CHUNK)], i2_v)
    pltpu.sync_copy(r1_h.at[pl.ds(base, _CHUNK)], r1_v)
    pltpu.sync_copy(r2_h.at[pl.ds(base, _CHUNK)], r2_v)
    pltpu.sync_copy(w1_h.at[pl.ds(base, _CHUNK)], w1_v)
    pltpu.sync_copy(w2_h.at[pl.ds(base, _CHUNK)], w2_v)
    for c in range(_NCH):
        sl = pl.ds(c * 16, 16)
        p1_v[sl] = plsc.load_gather(offs_v, [i1_v[sl]]) + r1_v[sl]
        p2_v[sl] = plsc.load_gather(offs_v, [i2_v[sl]]) + r2_v[sl]
    pltpu.sync_copy(p1_v, p1_h.at[pl.ds(base, _CHUNK)])
    pltpu.sync_copy(p2_v, p2_h.at[pl.ds(base, _CHUNK)])
    wd = []
    for c in range(_NCH):
        sl = pl.ds(c * 16, 16)
        wd.append(pltpu.async_copy(w1_v.at[sl], ws_h.at[p1_v[sl]], sem_w))
        wd.append(pltpu.async_copy(w2_v.at[sl], ws_h.at[p2_v[sl]], sem_w))
    def tok(c):
        return (base + c * 16
                + jax.lax.broadcasted_iota(jnp.int32, (16,), 0))

    g = pltpu.async_copy(flat_h.at[tok(0)], rows_v.at[0], sem_g)
    s_prev = []
    for c in range(_NCH):
        cur = c % 2
        sl = pl.ds(c * 16, 16)
        g.wait()
        for s in s_prev:
            s.wait()
        if c + 1 < _NCH:
            g = pltpu.async_copy(flat_h.at[tok(c + 1)], rows_v.at[1 - cur],
                                 sem_g)
        s_prev = [
            pltpu.async_copy(rows_v.at[cur], xs_h.at[p1_v[sl]], sem_s),
            pltpu.async_copy(rows_v.at[cur], xs_h.at[p2_v[sl]], sem_s),
        ]
    for s in s_prev:
        s.wait()
    for w in wd:
        w.wait()


def _dispatch(flat, i1, i2, r1, r2, w1, w2, offs16):
    return pl.kernel(
        _dispatch_body,
        out_type=[
            jax.ShapeDtypeStruct((_PAD, _D), jnp.float32),
            jax.ShapeDtypeStruct((_PAD, _WSW), jnp.float32),
            jax.ShapeDtypeStruct((_T,), jnp.int32),
            jax.ShapeDtypeStruct((_T,), jnp.int32),
        ],
        mesh=plsc.VectorSubcoreMesh(core_axis_name="c", subcore_axis_name="s"),
        compiler_params=pltpu.CompilerParams(needs_layout_passes=False),
        scratch_types=[
            pltpu.VMEM((16,), jnp.int32),
            pltpu.VMEM((_CHUNK,), jnp.int32),
            pltpu.VMEM((_CHUNK,), jnp.int32),
            pltpu.VMEM((_CHUNK,), jnp.int32),
            pltpu.VMEM((_CHUNK,), jnp.int32),
            pltpu.VMEM((_CHUNK, _WSW), jnp.float32),
            pltpu.VMEM((_CHUNK, _WSW), jnp.float32),
            pltpu.VMEM((_CHUNK,), jnp.int32),
            pltpu.VMEM((_CHUNK,), jnp.int32),
            pltpu.VMEM((2, 16, _D), jnp.float32),
            pltpu.SemaphoreType.DMA,
            pltpu.SemaphoreType.DMA,
            pltpu.SemaphoreType.DMA,
        ],
    )(flat, i1, i2, r1, r2, w1, w2, offs16)


def _gmm_body(be_ref, ntot_ref, xs_ref, g_ref, u_ref, d_ref, ws_ref, o_ref):
    b = pl.program_id(0)
    c = pl.program_id(1)

    @pl.when(b < ntot_ref[0])
    def _():
        x = xs_ref[...]
        g = jax.lax.dot_general(x, g_ref[0], (((1,), (1,)), ((), ())),
                                preferred_element_type=jnp.float32)
        u = jax.lax.dot_general(x, u_ref[0], (((1,), (1,)), ((), ())),
                                preferred_element_type=jnp.float32)
        h = (g * jax.nn.sigmoid(g)) * u
        part = jax.lax.dot_general(h, d_ref[0], (((1,), (1,)), ((), ())),
                                   preferred_element_type=jnp.float32)
        contrib = part * ws_ref[:, :1]

        @pl.when(c == 0)
        def _():
            o_ref[...] = contrib

        @pl.when(c != 0)
        def _():
            o_ref[...] += contrib


def _bb(b, ntot):
    return jnp.minimum(b, ntot[0] - 1)


def _cc(b, c, ntot):
    return jnp.where(b < ntot[0], c, _NC - 1)


def _gmm(be16, ntot1, xs, gate_w, up_w, down_w, ws):
    grid_spec = pltpu.PrefetchScalarGridSpec(
        num_scalar_prefetch=2,
        grid=(_NBLK, _NC),
        in_specs=[
            pl.BlockSpec((_R, _D), lambda b, c, be, nt: (_bb(b, nt), 0)),
            pl.BlockSpec((1, _FC, _D),
                         lambda b, c, be, nt: (be[_bb(b, nt)],
                                               _cc(b, c, nt), 0)),
            pl.BlockSpec((1, _FC, _D),
                         lambda b, c, be, nt: (be[_bb(b, nt)],
                                               _cc(b, c, nt), 0)),
            pl.BlockSpec((1, _D, _FC),
                         lambda b, c, be, nt: (be[_bb(b, nt)], 0,
                                               _cc(b, c, nt))),
            pl.BlockSpec((_R, _WSW), lambda b, c, be, nt: (_bb(b, nt), 0)),
        ],
        out_specs=pl.BlockSpec((_R, _D),
                               lambda b, c, be, nt: (_bb(b, nt), 0)),
    )
    return pl.pallas_call(
        _gmm_body,
        grid_spec=grid_spec,
        out_shape=jax.ShapeDtypeStruct((_PAD, _D), jnp.float32),
        compiler_params=pltpu.CompilerParams(
            dimension_semantics=("arbitrary", "arbitrary")),
    )(be16, ntot1, xs, gate_w, up_w, down_w, ws)


def _combine_body(ys_h, p1_h, p2_h, out_h,
                  p1_v, p2_v, a_v, b_v, o_v, sem_a, sem_b, sem_o):
    wid = lax.axis_index("s") * _SC_NC + lax.axis_index("c")
    base = wid * _CHUNK
    pltpu.sync_copy(p1_h.at[pl.ds(base, _CHUNK)], p1_v)
    pltpu.sync_copy(p2_h.at[pl.ds(base, _CHUNK)], p2_v)
    nch = _CHUNK // _CW
    ga = pltpu.async_copy(ys_h.at[p1_v.at[pl.ds(0, _CW)]], a_v.at[0], sem_a)
    gb = pltpu.async_copy(ys_h.at[p2_v.at[pl.ds(0, _CW)]], b_v.at[0], sem_b)
    od = [None, None]
    for c in range(nch):
        cur = c % 2
        ga.wait()
        gb.wait()
        if c + 1 < nch:
            sl = pl.ds((c + 1) * _CW, _CW)
            ga = pltpu.async_copy(ys_h.at[p1_v.at[sl]], a_v.at[1 - cur],
                                  sem_a)
            gb = pltpu.async_copy(ys_h.at[p2_v.at[sl]], b_v.at[1 - cur],
                                  sem_b)
        if od[cur] is not None:
            od[cur].wait()
        for r in range(_CW):
            @plsc.parallel_loop(0, _D // 16, unroll=8)
            def _(j, cur=cur, r=r):
                cs = pl.ds(j * 16, 16)
                o_v[cur, r, cs] = a_v[cur, r, cs] + b_v[cur, r, cs]

        od[cur] = pltpu.async_copy(o_v.at[cur],
                                   out_h.at[pl.ds(base + c * _CW, _CW)],
                                   sem_o)
    for d in od:
        if d is not None:
            d.wait()


def _combine(ys, p1, p2):
    return pl.kernel(
        _combine_body,
        out_type=jax.ShapeDtypeStruct((_T, _D), jnp.float32),
        mesh=plsc.VectorSubcoreMesh(core_axis_name="c", subcore_axis_name="s"),
        compiler_params=pltpu.CompilerParams(needs_layout_passes=False),
        scratch_types=[
            pltpu.VMEM((_CHUNK,), jnp.int32),
            pltpu.VMEM((_CHUNK,), jnp.int32),
            pltpu.VMEM((2, _CW, _D), jnp.float32),
            pltpu.VMEM((2, _CW, _D), jnp.float32),
            pltpu.VMEM((2, _CW, _D), jnp.float32),
            pltpu.SemaphoreType.DMA,
            pltpu.SemaphoreType.DMA,
            pltpu.SemaphoreType.DMA,
        ],
    )(ys, p1, p2)


def kernel(hidden_states, router_norm_w, router_w, gate_w, up_w, down_w):
    b, s, d = hidden_states.shape
    flat = hidden_states.reshape(b * s, d)
    i1, i2, r1, r2, w1, w2, counts = _router(flat, router_norm_w, router_w)
    cnt = counts[0]
    ntil = (cnt + _R - 1) // _R
    bounds = jnp.cumsum(ntil).astype(jnp.int32)
    boff = jnp.concatenate([jnp.zeros((1,), jnp.int32), bounds[:-1]])
    ntot1 = bounds[-1:]
    be16 = jnp.minimum(
        jnp.sum((bounds[None, :] <= jnp.arange(_NBLK)[:, None]).astype(
            jnp.int32), axis=1), _NE - 1).astype(jnp.int32)
    offs16 = jnp.zeros((16,), jnp.int32).at[:_NE].set(boff * _R)
    xs, ws, p1, p2 = _dispatch(flat, i1.reshape(_T), i2.reshape(_T),
                               r1.reshape(_T), r2.reshape(_T), w1, w2, offs16)
    ys = _gmm(be16, ntot1, xs, gate_w, up_w, down_w, ws)
    out = _combine(ys, p1, p2)
    aux = jnp.array(0.0, dtype=jnp.float32)
    return (out.reshape(b, s, d), aux)
